# baseline (speedup 1.0000x reference)
import functools

import jax
import jax.numpy as jnp
from jax import lax
from jax.experimental import pallas as pl
from jax.experimental.pallas import tpu as pltpu
from jax.experimental.pallas import tpu_sc as plsc

L = 16


def _matvec_relu_matvec(src_scr, w1t, b1, w2t, b2, d_in, d_mid, d_out):
    n_mid = d_mid // L
    n_out = d_out // L

    def body1(jc, acc):
        vchunk = src_scr[pl.ds(jc * L, L)]
        for jl in range(L):
            bj = jnp.full((L,), vchunk[jl], jnp.float32)
            acc = tuple(acc[o] + bj * w1t[jc * L + jl, pl.ds(o * L, L)]
                        for o in range(n_mid))
        return acc

    acc0 = tuple(b1[pl.ds(o * L, L)] for o in range(n_mid))
    h = lax.fori_loop(0, d_in // L, body1, acc0, unroll=True)
    h = tuple(jnp.maximum(h[o], 0.0) for o in range(n_mid))

    acc = tuple(b2[pl.ds(o * L, L)] for o in range(n_out))
    for jc in range(n_mid):
        vchunk = h[jc]
        for jl in range(L):
            bj = jnp.full((L,), vchunk[jl], jnp.float32)
            acc = tuple(acc[o] + bj * w2t[jc * L + jl, pl.ds(o * L, L)]
                        for o in range(n_out))
    return acc


def _sc_body(x0_h, w1s_h, b1s_h, w2s_h, b2s_h, w1c_h, b1c_h, w2c_h, b2c_h,
             out_h, wa, ba, wb, bb, v_scr, o_scr, *, n, latent, d_out):
    cid = lax.axis_index("c")
    sid = lax.axis_index("s")
    n_lat = latent // L
    n_o = d_out // L

    @pl.when(jnp.logical_and(cid == 0, sid == 0))
    def _():
        pltpu.sync_copy(x0_h, v_scr)
        pltpu.sync_copy(w1s_h, wa)
        pltpu.sync_copy(b1s_h, ba)
        pltpu.sync_copy(w2s_h, wb)
        pltpu.sync_copy(b2s_h, bb)

        def spread_step(_, carry):
            v = _matvec_relu_matvec(v_scr, wa, ba, wb, bb,
                                    latent, 2 * latent, latent)
            for c in range(n_lat):
                v_scr[pl.ds(c * L, L)] = v[c]
            return carry

        lax.fori_loop(0, n - 1, spread_step, 0)

    @pl.when(jnp.logical_and(cid == 1, sid == 0))
    def _():
        pltpu.sync_copy(w1c_h, wa)
        pltpu.sync_copy(b1c_h, ba)
        pltpu.sync_copy(w2c_h, wb)
        pltpu.sync_copy(b2c_h, bb)

        def collect_step(ii, pos):
            @pl.when(pos != ii)
            def _():
                for c in range(n_lat):
                    v_scr[pl.ds(c * L, L)] = jnp.zeros((L,), jnp.float32)

            v = _matvec_relu_matvec(v_scr, wa, ba, wb, bb,
                                    latent, 2 * d_out, d_out)
            for c in range(n_o):
                v_scr[pl.ds(c * L, L)] = v[c]
            return ii - 1

        pos = lax.fori_loop(1, n, collect_step, n - 1)

        for c in range(n_o):
            o_scr[pl.ds(c * L, L)] = v_scr[pl.ds(c * L, L)]

        @pl.when(pos != 0)
        def _():
            for c in range(n_o):
                o_scr[pl.ds(c * L, L)] = jnp.zeros((L,), jnp.float32)

        pltpu.sync_copy(o_scr, out_h)


def kernel(x, num_node, edge_index, W1s, b1s, W2s, b2s, W1c, b1c, W2c, b2c):
    del num_node, edge_index
    n = x.shape[0]
    assert n > 2
    latent = W2s.shape[0]
    d_out = W2c.shape[0]
    x0 = jnp.pad(x[0, :], (0, latent - x.shape[1]))

    mesh = plsc.VectorSubcoreMesh(core_axis_name="c", subcore_axis_name="s")
    body = functools.partial(_sc_body, n=n, latent=latent, d_out=d_out)
    run = pl.kernel(
        body,
        out_type=jax.ShapeDtypeStruct((d_out,), jnp.float32),
        mesh=mesh,
        scratch_types=[
            pltpu.VMEM((latent, 2 * latent), jnp.float32),
            pltpu.VMEM((2 * latent,), jnp.float32),
            pltpu.VMEM((2 * latent, latent), jnp.float32),
            pltpu.VMEM((latent,), jnp.float32),
            pltpu.VMEM((latent,), jnp.float32),
            pltpu.VMEM((d_out,), jnp.float32),
        ],
    )
    return run(x0, W1s.T, b1s, W2s.T, b2s, W1c.T, b1c, W2c.T, b2c)


# speedup vs baseline: 1.5443x; 1.5443x over previous
import functools

import jax
import jax.numpy as jnp
from jax import lax
from jax.experimental import pallas as pl
from jax.experimental.pallas import tpu as pltpu
from jax.experimental.pallas import tpu_sc as plsc

L = 16


def _matvec_relu_matvec(src_scr, w1t, b1, w2t, b2, d_in, d_mid, d_out, h_scr):
    n_mid = d_mid // L
    n_out = d_out // L

    def body1(jc, acc):
        vchunk = src_scr[pl.ds(jc * L, L)]
        for jl in range(L):
            bj = jnp.full((L,), vchunk[jl], jnp.float32)
            acc = tuple(acc[o] + bj * w1t[jc * L + jl, pl.ds(o * L, L)]
                        for o in range(n_mid))
        return acc

    acc0 = tuple(b1[pl.ds(o * L, L)] for o in range(n_mid))
    h = lax.fori_loop(0, d_in // L, body1, acc0, unroll=2)
    for o in range(n_mid):
        h_scr[pl.ds(o * L, L)] = jnp.maximum(h[o], 0.0)

    def body2(jc, acc):
        vchunk = h_scr[pl.ds(jc * L, L)]
        for jl in range(L):
            bj = jnp.full((L,), vchunk[jl], jnp.float32)
            acc = tuple(acc[o] + bj * w2t[jc * L + jl, pl.ds(o * L, L)]
                        for o in range(n_out))
        return acc

    acc1 = tuple(b2[pl.ds(o * L, L)] for o in range(n_out))
    return lax.fori_loop(0, d_mid // L, body2, acc1, unroll=2)


def _sc_body(x0_h, w1s_h, b1s_h, w2s_h, b2s_h, w1c_h, b1c_h, w2c_h, b2c_h,
             out_h, wa, ba, wb, bb, v_scr, h_scr, o_scr, *, n, latent, d_out):
    cid = lax.axis_index("c")
    sid = lax.axis_index("s")
    n_lat = latent // L
    n_o = d_out // L

    @pl.when(jnp.logical_and(cid == 0, sid == 0))
    def _():
        pltpu.sync_copy(x0_h, v_scr)
        pltpu.sync_copy(w1s_h, wa)
        pltpu.sync_copy(b1s_h, ba)
        pltpu.sync_copy(w2s_h, wb)
        pltpu.sync_copy(b2s_h, bb)

        def spread_step(_, carry):
            v = _matvec_relu_matvec(v_scr, wa, ba, wb, bb,
                                    latent, 2 * latent, latent, h_scr)
            for c in range(n_lat):
                v_scr[pl.ds(c * L, L)] = v[c]
            return carry

        lax.fori_loop(0, n - 1, spread_step, 0)

    @pl.when(jnp.logical_and(cid == 1, sid == 0))
    def _():
        pltpu.sync_copy(w1c_h, wa)
        pltpu.sync_copy(b1c_h, ba)
        pltpu.sync_copy(w2c_h, wb)
        pltpu.sync_copy(b2c_h, bb)

        def collect_step(ii, pos):
            @pl.when(pos != ii)
            def _():
                for c in range(n_lat):
                    v_scr[pl.ds(c * L, L)] = jnp.zeros((L,), jnp.float32)

            v = _matvec_relu_matvec(v_scr, wa, ba, wb, bb,
                                    latent, 2 * d_out, d_out, h_scr)
            for c in range(n_o):
                v_scr[pl.ds(c * L, L)] = v[c]
            return ii - 1

        pos = lax.fori_loop(1, n, collect_step, n - 1)

        for c in range(n_o):
            o_scr[pl.ds(c * L, L)] = v_scr[pl.ds(c * L, L)]

        @pl.when(pos != 0)
        def _():
            for c in range(n_o):
                o_scr[pl.ds(c * L, L)] = jnp.zeros((L,), jnp.float32)

        pltpu.sync_copy(o_scr, out_h)


def kernel(x, num_node, edge_index, W1s, b1s, W2s, b2s, W1c, b1c, W2c, b2c):
    del num_node, edge_index
    n = x.shape[0]
    assert n > 2
    latent = W2s.shape[0]
    d_out = W2c.shape[0]
    x0 = jnp.pad(x[0, :], (0, latent - x.shape[1]))

    mesh = plsc.VectorSubcoreMesh(core_axis_name="c", subcore_axis_name="s")
    body = functools.partial(_sc_body, n=n, latent=latent, d_out=d_out)
    run = pl.kernel(
        body,
        out_type=jax.ShapeDtypeStruct((d_out,), jnp.float32),
        mesh=mesh,
        scratch_types=[
            pltpu.VMEM((latent, 2 * latent), jnp.float32),
            pltpu.VMEM((2 * latent,), jnp.float32),
            pltpu.VMEM((2 * latent, latent), jnp.float32),
            pltpu.VMEM((latent,), jnp.float32),
            pltpu.VMEM((latent,), jnp.float32),
            pltpu.VMEM((2 * latent,), jnp.float32),
            pltpu.VMEM((d_out,), jnp.float32),
        ],
    )
    return run(x0, W1s.T, b1s, W2s.T, b2s, W1c.T, b1c, W2c.T, b2c)


# speedup vs baseline: 2.1862x; 1.4157x over previous
import functools

import jax
import jax.numpy as jnp
from jax import lax
from jax.experimental import pallas as pl
from jax.experimental.pallas import tpu as pltpu
from jax.experimental.pallas import tpu_sc as plsc

L = 16


def _matvec_relu_matvec(src_scr, w1t, b1, w2t, b2, d_in, d_mid, d_out, h_scr):
    n_mid = d_mid // L
    n_out = d_out // L

    def body1(jc, acc):
        vchunk = src_scr[pl.ds(jc * L, L)]
        for jl in range(L):
            bj = jnp.full((L,), vchunk[jl], jnp.float32)
            acc = tuple(acc[o] + bj * w1t[jc * L + jl, pl.ds(o * L, L)]
                        for o in range(n_mid))
        return acc

    acc0 = tuple(b1[pl.ds(o * L, L)] for o in range(n_mid))
    h = lax.fori_loop(0, d_in // L, body1, acc0)
    for o in range(n_mid):
        h_scr[pl.ds(o * L, L)] = jnp.maximum(h[o], 0.0)

    def body2(jc, acc):
        vchunk = h_scr[pl.ds(jc * L, L)]
        for jl in range(L):
            bj = jnp.full((L,), vchunk[jl], jnp.float32)
            acc = tuple(acc[o] + bj * w2t[jc * L + jl, pl.ds(o * L, L)]
                        for o in range(n_out))
        return acc

    acc1 = tuple(b2[pl.ds(o * L, L)] for o in range(n_out))
    return lax.fori_loop(0, d_mid // L, body2, acc1)


def _sc_body(x0_h, w1s_h, b1s_h, w2s_h, b2s_h, w1c_h, b1c_h, w2c_h, b2c_h,
             out_h, wa, ba, wb, bb, v_scr, h_scr, o_scr, *, n, latent, d_out):
    cid = lax.axis_index("c")
    sid = lax.axis_index("s")
    n_lat = latent // L
    n_o = d_out // L
    is_spread = cid == 0

    @pl.when(sid == 0)
    def _():
        @pl.when(is_spread)
        def _():
            pltpu.sync_copy(x0_h, v_scr)
            pltpu.sync_copy(w1s_h, wa)
            pltpu.sync_copy(b1s_h, ba)
            pltpu.sync_copy(w2s_h, wb)
            pltpu.sync_copy(b2s_h, bb)

        @pl.when(jnp.logical_not(is_spread))
        def _():
            pltpu.sync_copy(w1c_h, wa)
            pltpu.sync_copy(b1c_h, ba)
            pltpu.sync_copy(w2c_h, wb)
            pltpu.sync_copy(b2c_h, bb)

        dirn = 1 - 2 * cid
        pos0 = 1 + (n - 2) * cid

        def step(ii, pos):
            @pl.when(pos != ii)
            def _():
                for c in range(n_lat):
                    v_scr[pl.ds(c * L, L)] = jnp.zeros((L,), jnp.float32)

            v = _matvec_relu_matvec(v_scr, wa, ba, wb, bb,
                                    latent, 2 * latent, d_out, h_scr)
            for c in range(n_o):
                v_scr[pl.ds(c * L, L)] = v[c]
            return ii + dirn

        pos = lax.fori_loop(1, n, step, pos0)

        @pl.when(jnp.logical_not(is_spread))
        def _():
            for c in range(n_o):
                o_scr[pl.ds(c * L, L)] = v_scr[pl.ds(c * L, L)]

            @pl.when(pos != 0)
            def _():
                for c in range(n_o):
                    o_scr[pl.ds(c * L, L)] = jnp.zeros((L,), jnp.float32)

            pltpu.sync_copy(o_scr, out_h)


def kernel(x, num_node, edge_index, W1s, b1s, W2s, b2s, W1c, b1c, W2c, b2c):
    del num_node, edge_index
    n = x.shape[0]
    assert n > 2
    latent = W2s.shape[0]
    d_out = W2c.shape[0]
    assert latent == d_out
    x0 = jnp.pad(x[0, :], (0, latent - x.shape[1]))

    mesh = plsc.VectorSubcoreMesh(core_axis_name="c", subcore_axis_name="s")
    body = functools.partial(_sc_body, n=n, latent=latent, d_out=d_out)
    run = pl.kernel(
        body,
        out_type=jax.ShapeDtypeStruct((d_out,), jnp.float32),
        mesh=mesh,
        scratch_types=[
            pltpu.VMEM((latent, 2 * latent), jnp.float32),
            pltpu.VMEM((2 * latent,), jnp.float32),
            pltpu.VMEM((2 * latent, latent), jnp.float32),
            pltpu.VMEM((latent,), jnp.float32),
            pltpu.VMEM((latent,), jnp.float32),
            pltpu.VMEM((2 * latent,), jnp.float32),
            pltpu.VMEM((d_out,), jnp.float32),
        ],
    )
    return run(x0, W1s.T, b1s, W2s.T, b2s, W1c.T, b1c, W2c.T, b2c)


# speedup vs baseline: 2.9132x; 1.3325x over previous
import functools

import jax
import jax.numpy as jnp
from jax import lax
from jax.experimental import pallas as pl
from jax.experimental.pallas import tpu as pltpu
from jax.experimental.pallas import tpu_sc as plsc

L = 16


def _sc_body(x0_h, w1s_h, b1s_h, w2s_h, b2s_h, w1c_h, b1c_h, w2c_h, b2c_h,
             out_h, wa, ba, wb, bb, v_scr, h_scr, p_scr, q_scr, o_scr, shared,
             *, n, d):
    cid = lax.axis_index("c")
    sid = lax.axis_index("s")
    nc = d // L
    is_spread = cid == 0
    is_worker = sid < 2

    @pl.when(jnp.logical_and(is_worker, is_spread))
    def _():
        pltpu.sync_copy(x0_h, v_scr)
        pltpu.sync_copy(w1s_h.at[sid], wa)
        pltpu.sync_copy(b1s_h.at[sid], ba)
        pltpu.sync_copy(w2s_h.at[sid], wb)
        pltpu.sync_copy(b2s_h, bb)

    @pl.when(jnp.logical_and(is_worker, jnp.logical_not(is_spread)))
    def _():
        pltpu.sync_copy(w1c_h.at[sid], wa)
        pltpu.sync_copy(b1c_h.at[sid], ba)
        pltpu.sync_copy(w2c_h.at[sid], wb)
        pltpu.sync_copy(b2c_h, bb)

    @pl.when(jnp.logical_and(is_worker, sid == 1))
    def _():
        for c in range(nc):
            bb[pl.ds(c * L, L)] = jnp.zeros((L,), jnp.float32)

    dirn = 1 - 2 * cid
    pos0 = 1 + (n - 2) * cid

    def step(ii, pos):
        @pl.when(is_worker)
        def _():
            @pl.when(pos != ii)
            def _():
                for c in range(nc):
                    v_scr[pl.ds(c * L, L)] = jnp.zeros((L,), jnp.float32)

            def body1(jc, acc):
                vchunk = v_scr[pl.ds(jc * L, L)]
                for jl in range(L):
                    bj = jnp.full((L,), vchunk[jl], jnp.float32)
                    acc = tuple(acc[o] + bj * wa[jc * L + jl, pl.ds(o * L, L)]
                                for o in range(nc))
                return acc

            h = lax.fori_loop(0, nc, body1,
                              tuple(ba[pl.ds(o * L, L)] for o in range(nc)))
            for o in range(nc):
                h_scr[pl.ds(o * L, L)] = jnp.maximum(h[o], 0.0)

            def body2(jc, acc):
                vchunk = h_scr[pl.ds(jc * L, L)]
                for jl in range(L):
                    bj = jnp.full((L,), vchunk[jl], jnp.float32)
                    acc = tuple(acc[o] + bj * wb[jc * L + jl, pl.ds(o * L, L)]
                                for o in range(nc))
                return acc

            part = lax.fori_loop(0, nc, body2,
                                 tuple(bb[pl.ds(o * L, L)] for o in range(nc)))
            for o in range(nc):
                p_scr[pl.ds(o * L, L)] = part[o]
            pltpu.sync_copy(p_scr, shared.at[ii % 2, sid])

        plsc.subcore_barrier()

        @pl.when(is_worker)
        def _():
            pltpu.sync_copy(shared.at[ii % 2, 1 - sid], q_scr)
            for c in range(nc):
                v_scr[pl.ds(c * L, L)] = (p_scr[pl.ds(c * L, L)]
                                          + q_scr[pl.ds(c * L, L)])

        return ii + dirn

    pos = lax.fori_loop(1, n, step, pos0)

    @pl.when(jnp.logical_and(cid == 1, sid == 0))
    def _():
        for c in range(nc):
            o_scr[pl.ds(c * L, L)] = v_scr[pl.ds(c * L, L)]

        @pl.when(pos != 0)
        def _():
            for c in range(nc):
                o_scr[pl.ds(c * L, L)] = jnp.zeros((L,), jnp.float32)

        pltpu.sync_copy(o_scr, out_h)


def kernel(x, num_node, edge_index, W1s, b1s, W2s, b2s, W1c, b1c, W2c, b2c):
    del num_node, edge_index
    n = x.shape[0]
    assert n > 2
    d = W2s.shape[0]
    assert W2c.shape[0] == d and W1s.shape[0] == 2 * d and W1c.shape[0] == 2 * d
    x0 = jnp.pad(x[0, :], (0, d - x.shape[1]))

    def split1(w1):
        return w1.T.reshape(d, 2, d).transpose(1, 0, 2)

    def split2(w2):
        return w2.T.reshape(2, d, d)

    mesh = plsc.VectorSubcoreMesh(core_axis_name="c", subcore_axis_name="s")
    body = functools.partial(_sc_body, n=n, d=d)
    run = pl.kernel(
        body,
        out_type=jax.ShapeDtypeStruct((d,), jnp.float32),
        mesh=mesh,
        scratch_types=[
            pltpu.VMEM((d, d), jnp.float32),
            pltpu.VMEM((d,), jnp.float32),
            pltpu.VMEM((d, d), jnp.float32),
            pltpu.VMEM((d,), jnp.float32),
            pltpu.VMEM((d,), jnp.float32),
            pltpu.VMEM((d,), jnp.float32),
            pltpu.VMEM((d,), jnp.float32),
            pltpu.VMEM((d,), jnp.float32),
            pltpu.VMEM((d,), jnp.float32),
            pltpu.VMEM_SHARED((2, 2, d), jnp.float32),
        ],
    )
    return run(x0,
               split1(W1s), b1s.reshape(2, d), split2(W2s), b2s,
               split1(W1c), b1c.reshape(2, d), split2(W2c), b2c)
